# baseline (speedup 1.0000x reference)
import functools

import jax
import jax.numpy as jnp
from jax import lax
from jax.experimental import pallas as pl
from jax.experimental.pallas import tpu as pltpu
from jax.experimental.pallas import tpu_sc as plsc

DIM = 64
B = 16384
LAMBD = 0.1
NW = 32
CHUNK = B // NW
SEG = 128
NSEG = CHUNK // SEG
BLK = 1024
GRID = B // BLK

_mesh = plsc.VectorSubcoreMesh(core_axis_name="c", subcore_axis_name="s",
                               num_cores=2, num_subcores=16)


def _make_gather(rows):
    @functools.partial(
        pl.kernel,
        mesh=_mesh,
        compiler_params=pltpu.CompilerParams(needs_layout_passes=False,
                                             use_tc_tiling_on_sc=True),
        out_type=jax.ShapeDtypeStruct((B, 2 * DIM), jnp.float32),
        scratch_types=[
            pltpu.VMEM((NSEG, SEG), jnp.int32),
            pltpu.VMEM((CHUNK, 2 * DIM), jnp.float32),
            pltpu.SemaphoreType.DMA,
        ],
        name=f"sc_gather_{rows}",
    )
    def gather(idx_hbm, tab_hbm, out_hbm, idx_v, rows_v, sem):
        wid = lax.axis_index("s") * 2 + lax.axis_index("c")
        base = wid * CHUNK
        pltpu.sync_copy(idx_hbm.at[pl.ds(wid * NSEG, NSEG)], idx_v)
        handles = [
            pltpu.async_copy(tab_hbm.at[idx_v.at[k]],
                             rows_v.at[pl.ds(k * SEG, SEG)], sem)
            for k in range(NSEG)
        ]
        for h in handles:
            h.wait()
        pltpu.sync_copy(rows_v, out_hbm.at[pl.ds(base, CHUNK)])

    return gather


_gather_node = _make_gather(500000)
_gather_base = _make_gather(50000)


def _half_dots(x, y):
    xl, xr = x[:, :DIM], x[:, DIM:]
    yl, yr = y[:, :DIM], y[:, DIM:]
    return (jnp.sum(xl * yl, axis=1, keepdims=True),
            jnp.sum(xl * yr, axis=1, keepdims=True),
            jnp.sum(xr * yl, axis=1, keepdims=True),
            jnp.sum(xr * yr, axis=1, keepdims=True))


def _select4(ll, lr, rl, rr, pa, pb):
    qa, qb = 1.0 - pa, 1.0 - pb
    return qa * qb * ll + qa * pb * lr + pa * qb * rl + pa * pb * rr


def _loss_body(a_ref, b_ref, c_ref, d_ref,
               pa_ref, pb_ref, pc_ref, pd_ref, t_ref,
               o_ref, acc_ref):
    g = pl.program_id(0)

    a, b = a_ref[...], b_ref[...]
    pa, pb = pa_ref[...], pb_ref[...]
    sll, slr, srl, srr = _half_dots(a, b)
    s = _select4(sll, slr, srl, srr, pa, pb)
    al, ar = a[:, :DIM], a[:, DIM:]
    bl, br = b[:, :DIM], b[:, DIM:]
    na = _select4(jnp.sum(al * al, axis=1, keepdims=True), 0.0, 0.0,
                  jnp.sum(ar * ar, axis=1, keepdims=True), pa, pa)
    nb = _select4(jnp.sum(bl * bl, axis=1, keepdims=True), 0.0, 0.0,
                  jnp.sum(br * br, axis=1, keepdims=True), pb, pb)
    na = jnp.maximum(jnp.sqrt(na), 1e-12)
    nb = jnp.maximum(jnp.sqrt(nb), 1e-12)
    scores = jax.nn.sigmoid(s / (na * nb))
    t = t_ref[...]
    ml = t * jnp.log(scores) + (1.0 - t) * jnp.log(1.0 - scores)

    c, d = c_ref[...], d_ref[...]
    rll, rlr, rrl, rrr = _half_dots(c, d)
    r = _select4(rll, rlr, rrl, rrr, pc_ref[...], pd_ref[...])
    rl_ = jnp.log(jax.nn.sigmoid(jnp.clip(r, -15.0, 15.0)))

    partial = jnp.sum(ml) + LAMBD * jnp.sum(rl_)

    @pl.when(g == 0)
    def _():
        acc_ref[0, 0] = 0.0

    acc_ref[0, 0] += partial

    @pl.when(g == GRID - 1)
    def _():
        o_ref[...] = jnp.reshape(-acc_ref[0, 0] / B, (1, 1))


_finish = pl.pallas_call(
    _loss_body,
    grid=(GRID,),
    in_specs=[pl.BlockSpec((BLK, 2 * DIM), lambda g: (g, 0))] * 4
    + [pl.BlockSpec((BLK, 1), lambda g: (g, 0))] * 5,
    out_specs=pl.BlockSpec((1, 1), lambda g: (0, 0)),
    out_shape=jax.ShapeDtypeStruct((1, 1), jnp.float32),
    scratch_shapes=[pltpu.SMEM((1, 1), jnp.float32)],
)


def _split(idx):
    idx = idx.astype(jnp.int32)
    phys = (idx >> 1).reshape(NW * NSEG, SEG)
    par = (idx & 1).astype(jnp.float32).reshape(B, 1)
    return phys, par


@jax.jit
def kernel(sources, contexts, targets, personas, pure_sources,
           node_embedding, node_noise_embedding, base_node_embedding):
    srcp, pa = _split(sources)
    ctxp, pb = _split(contexts)
    purep, pc = _split(pure_sources)
    perp, pd = _split(personas)
    node2 = node_embedding.reshape(-1, 2 * DIM)
    noise2 = node_noise_embedding.reshape(-1, 2 * DIM)
    base2 = base_node_embedding.reshape(-1, 2 * DIM)
    rows_src = _gather_node(srcp, node2)
    rows_ctx = _gather_node(ctxp, noise2)
    rows_pure = _gather_node(purep, node2)
    rows_per = _gather_base(perp, base2)
    out = _finish(rows_src, rows_ctx, rows_pure, rows_per,
                  pa, pb, pc, pd, targets.reshape(B, 1))
    return out.reshape(())
